# baseline (speedup 1.0000x reference)
import dataclasses
import functools
import math

import jax
import jax.numpy as jnp
from jax import lax
from jax.experimental import pallas as pl
from jax.experimental.pallas import tpu as pltpu
from jax.experimental.pallas import tpu_sc as plsc

NUM_OBJ = 1000
NUM_REL = 64
BATCH = 16384
EPS = 1e-3
LOG_PSB = math.log(1e-3)
LOG_BG = math.log(1e-3)

NC, NS, L = 2, 16, 16
NW = NC * NS
ROWS_PER_W = BATCH // NW
GCHUNK = 128
NCHUNK = ROWS_PER_W // GCHUNK


def _sc_gather_body(obj_hbm, fg_hbm, out_hbm, op_v, idx_v, rows_v, sem):
    wid = lax.axis_index("s") * NC + lax.axis_index("c")
    base = wid * ROWS_PER_W
    pltpu.sync_copy(obj_hbm.at[pl.ds(base, ROWS_PER_W)], op_v)
    lanes = lax.iota(jnp.int32, L)
    zeros = lanes * 0
    ones = zeros + 1
    for t in range(ROWS_PER_W // L):
        rows = t * L + lanes
        a = plsc.load_gather(op_v, [rows, zeros])
        b = plsc.load_gather(op_v, [rows, ones])
        idx_v[t * L // GCHUNK, pl.ds((t * L) % GCHUNK, L)] = a * NUM_OBJ + b
    copies = [
        pltpu.async_copy(
            fg_hbm.at[idx_v.at[j]],
            rows_v.at[pl.ds(j * GCHUNK, GCHUNK)],
            sem,
        )
        for j in range(NCHUNK)
    ]
    for c in copies:
        c.wait()
    pltpu.sync_copy(rows_v, out_hbm.at[pl.ds(base, ROWS_PER_W)])


def _sc_gather(obj_pair, fg_count):
    mesh = plsc.VectorSubcoreMesh(core_axis_name="c", subcore_axis_name="s")
    cp = pltpu.CompilerParams()
    if "needs_layout_passes" in pltpu.CompilerParams.__dataclass_fields__:
        cp = dataclasses.replace(cp, needs_layout_passes=False)
    cp = dataclasses.replace(cp, use_tc_tiling_on_sc=False)
    k = pl.kernel(
        _sc_gather_body,
        out_type=jax.ShapeDtypeStruct((BATCH, NUM_REL), jnp.float32),
        mesh=mesh,
        scratch_types=[
            pltpu.VMEM((ROWS_PER_W, 2), jnp.int32),
            pltpu.VMEM((NCHUNK, GCHUNK), jnp.int32),
            pltpu.VMEM((ROWS_PER_W, NUM_REL), jnp.float32),
            pltpu.SemaphoreType.DMA,
        ],
        compiler_params=cp,
    )
    return k(obj_pair, fg_count)


def _tc_fuse_body(counts_ref, pred_ref, out_ref):
    c = counts_ref[...]
    denom = jnp.sum(c, axis=1, keepdims=True) + EPS
    bias = jnp.log(c / denom + EPS)
    bias = jnp.where(c == 0.0, LOG_PSB, bias)
    col = lax.broadcasted_iota(jnp.int32, c.shape, 1)
    bias = jnp.where(col == 0, LOG_BG, bias)
    out_ref[...] = pred_ref[...] + bias


def _tc_fuse(counts, pred_dist):
    blk = 1024
    grid = BATCH // blk
    return pl.pallas_call(
        _tc_fuse_body,
        out_shape=jax.ShapeDtypeStruct((BATCH, NUM_REL), jnp.float32),
        grid=(grid,),
        in_specs=[
            pl.BlockSpec((blk, NUM_REL), lambda i: (i, 0)),
            pl.BlockSpec((blk, NUM_REL), lambda i: (i, 0)),
        ],
        out_specs=pl.BlockSpec((blk, NUM_REL), lambda i: (i, 0)),
    )(counts, pred_dist)


def kernel(pred_dist, gt, obj_pair, fg_count):
    del gt
    counts = _sc_gather(obj_pair, fg_count)
    return _tc_fuse(counts, pred_dist)
